# Initial kernel scaffold; baseline (speedup 1.0000x reference)
import jax
import jax.numpy as jnp
from jax.experimental import pallas as pl


def kernel(search_tokens, pred_uv):
    raise NotImplementedError("write your pallas kernel here")


# speedup vs baseline: 4.1975x; 4.1975x over previous
import functools

import jax
import jax.numpy as jnp
from jax.experimental import pallas as pl
from jax.experimental.pallas import tpu as pltpu
from jax.experimental.pallas import tpu_sc as plsc

_B = 128
_N = 1024
_D = 384
_K = 512
_GRID = 32

_NC = 2
_NS = 16
_NW = _NC * _NS
_BPW = _B // _NW
_CH = 128
_NCH = _BPW * _K // _CH


def _cumsum_lanes(x):
    c = x
    n = x.shape[1]
    s = 1
    while s < n:
        c = c + jnp.concatenate(
            [jnp.zeros((x.shape[0], s), c.dtype), c[:, : n - s]], axis=1)
        s *= 2
    return c


def _select_body(uv_ref, pos_ref):
    uv = jnp.clip(uv_ref[...], 0.0, 1.0)
    ux = uv[:, 0:1]
    uy = uv[:, 1:2]
    idx = jax.lax.broadcasted_iota(jnp.int32, (_B, _N), 1)
    cx = ((idx % _GRID).astype(jnp.float32) + 0.5) / float(_GRID)
    cy = ((idx // _GRID).astype(jnp.float32) + 0.5) / float(_GRID)
    dx = ux - cx
    dy = uy - cy
    dist = jnp.sqrt(dx * dx + dy * dy)
    bits = jax.lax.bitcast_convert_type(dist, jnp.int32)

    def bs_step(_, carry):
        lo, hi = carry
        mid = lo + (hi - lo) // 2
        cnt = jnp.sum((bits <= mid).astype(jnp.int32), axis=1, keepdims=True)
        ge = cnt >= _K
        return jnp.where(ge, lo, mid), jnp.where(ge, mid, hi)

    lo0 = jnp.full((_B, 1), -1, jnp.int32)
    hi0 = jnp.full((_B, 1), 0x7F800000, jnp.int32)
    _, t = jax.lax.fori_loop(0, 31, bs_step, (lo0, hi0))
    n_less = jnp.sum((bits < t).astype(jnp.int32), axis=1, keepdims=True)
    m = _K - n_less
    tie = bits == t
    tie_i = tie.astype(jnp.int32)
    tie_excl = _cumsum_lanes(tie_i) - tie_i
    keep = (bits < t) | (tie & (tie_excl < m))
    kc = _cumsum_lanes(keep.astype(jnp.int32))
    pos_ref[...] = jnp.where(keep, kc - 1, (_K - 1) + (idx + 1 - kc))


def _select(pred_uv):
    return pl.pallas_call(
        _select_body,
        out_shape=jax.ShapeDtypeStruct((_B, _N), jnp.int32),
    )(pred_uv)


def _sc_body(pos_hbm, tok_hbm, out_hbm, keep_hbm, rem_hbm,
             pos_v, comb_v, gidx_v, rows_a, rows_b, sem_a, sem_b):
    c = jax.lax.axis_index("c")
    s = jax.lax.axis_index("s")
    wid = s * _NC + c
    lane = jax.lax.broadcasted_iota(jnp.int32, (16,), 0)

    for q in range(_BPW):
        b = wid * _BPW + q
        pltpu.sync_copy(pos_hbm.at[b], pos_v)

        def compact(i, carry, q=q, b=b):
            p = pos_v[pl.ds(i * 16, 16)]
            vals = i * 16 + lane
            plsc.store_scatter(comb_v, [p], vals)
            mk = p < _K
            gslot = jnp.where(mk, q * _K + p, 0)
            plsc.store_scatter(gidx_v, [gslot], vals + b * _N, mask=mk)
            return carry

        jax.lax.fori_loop(0, _N // 16, compact, 0)
        pltpu.sync_copy(comb_v.at[pl.ds(0, _K)], keep_hbm.at[b])
        pltpu.sync_copy(comb_v.at[pl.ds(_K, _K)], rem_hbm.at[b])

    bufs = (rows_a, rows_b)
    sems = (sem_a, sem_b)
    descs = [None, None]

    def start(j, slot):
        descs[slot] = pltpu.async_copy(
            tok_hbm.at[gidx_v.at[pl.ds(j * _CH, _CH)]], bufs[slot], sems[slot])

    base = wid * (_BPW * _K)
    start(0, 0)
    for j in range(_NCH):
        slot = j % 2
        if j + 1 < _NCH:
            start(j + 1, 1 - slot)
        descs[slot].wait()
        pltpu.sync_copy(bufs[slot], out_hbm.at[pl.ds(base + j * _CH, _CH)])


@functools.cache
def _sc_run():
    return pl.kernel(
        _sc_body,
        out_type=[
            jax.ShapeDtypeStruct((_B * _K, _D), jnp.float32),
            jax.ShapeDtypeStruct((_B, _K), jnp.int32),
            jax.ShapeDtypeStruct((_B, _K), jnp.int32),
        ],
        mesh=plsc.VectorSubcoreMesh(
            core_axis_name="c", subcore_axis_name="s",
            num_cores=_NC, num_subcores=_NS),
        scratch_types=[
            pltpu.VMEM((_N,), jnp.int32),
            pltpu.VMEM((_N,), jnp.int32),
            pltpu.VMEM((_BPW * _K,), jnp.int32),
            pltpu.VMEM((_CH, _D), jnp.float32),
            pltpu.VMEM((_CH, _D), jnp.float32),
            pltpu.SemaphoreType.DMA,
            pltpu.SemaphoreType.DMA,
        ],
        compiler_params=pltpu.CompilerParams(needs_layout_passes=False),
    )


def kernel(search_tokens, pred_uv):
    pos = _select(pred_uv)
    tok_flat = search_tokens.reshape(_B * _N, _D)
    out_flat, keep_idx, rem_idx = _sc_run()(pos, tok_flat)
    return (out_flat.reshape(_B, _K, _D), keep_idx, rem_idx, _K / float(_N))
